# Initial kernel scaffold; baseline (speedup 1.0000x reference)
import jax
import jax.numpy as jnp
from jax.experimental import pallas as pl


def kernel(x1, edge_index1, batch1, x2, edge_index2, batch2, W1, b1, W2, b2, W3, b3, Wa, Wt, Wblock, bt, Wfc, bfc, Wsc, bsc):
    raise NotImplementedError("write your pallas kernel here")


# speedup vs baseline: 17.2193x; 17.2193x over previous
import functools

import jax
import jax.numpy as jnp
from jax import lax
from jax.experimental import pallas as pl
from jax.experimental.pallas import tpu as pltpu
from jax.experimental.pallas import tpu_sc as plsc

N = 10000
E = 320000
D = 128
F1, F2, F3 = 64, 32, 16
T = 16
BN = 16
NC = 2
NT = 16
CH = 128
K = 160
EPAD = NC * NT * K * CH
RPT = 632
NP = NT * RPT
BLK = 2000
NB = N // BLK

_mesh = plsc.VectorSubcoreMesh(core_axis_name="c", subcore_axis_name="s")


def _sc_scatter(F):
    @functools.partial(
        pl.kernel,
        out_type=jax.ShapeDtypeStruct((NC * NP, F), jnp.float32),
        mesh=_mesh,
        compiler_params=pltpu.CompilerParams(use_tc_tiling_on_sc=False),
        scratch_types=[
            pltpu.VMEM_SHARED((NP, F), jnp.float32),
            pltpu.VMEM((K, CH), jnp.int32),
            pltpu.VMEM((K, CH), jnp.int32),
            pltpu.VMEM((CH, F), jnp.float32),
            pltpu.SemaphoreType.DMA,
        ],
    )
    def body(src_hbm, dst_hbm, hs_hbm, zeros_hbm, out_hbm, acc_sh, svm, dvm,
             rows, sem):
        c = lax.axis_index("c")
        t = lax.axis_index("s")
        r0 = t * RPT
        pltpu.sync_copy(zeros_hbm.at[pl.ds(r0, RPT)], acc_sh.at[pl.ds(r0, RPT)])
        eb = (c * NT + t) * K
        pltpu.sync_copy(src_hbm.at[pl.ds(eb, K)], svm)
        pltpu.sync_copy(dst_hbm.at[pl.ds(eb, K)], dvm)
        plsc.subcore_barrier()

        def chunk(k, carry):
            pltpu.async_copy(hs_hbm.at[svm.at[k]], rows, sem).wait()
            pltpu.sync_copy(rows, acc_sh.at[dvm.at[k]], add=True)
            return carry

        lax.fori_loop(0, K, chunk, 0)
        plsc.subcore_barrier()
        pltpu.sync_copy(acc_sh.at[pl.ds(r0, RPT)],
                        out_hbm.at[pl.ds(c * NP + r0, RPT)])

    return body


@functools.partial(
    pl.kernel,
    out_type=jax.ShapeDtypeStruct((NC * NP, 16), jnp.float32),
    mesh=_mesh,
    compiler_params=pltpu.CompilerParams(use_tc_tiling_on_sc=False),
    scratch_types=[
        pltpu.VMEM_SHARED((NP, 16), jnp.float32),
        pltpu.VMEM((K, CH), jnp.int32),
        pltpu.VMEM((CH, 16), jnp.float32),
    ],
)
def _sc_degree(dst_hbm, zeros_hbm, ones_hbm, out_hbm, acc_sh, dvm, ones_v):
    c = lax.axis_index("c")
    t = lax.axis_index("s")
    r0 = t * RPT
    pltpu.sync_copy(zeros_hbm.at[pl.ds(r0, RPT)], acc_sh.at[pl.ds(r0, RPT)])
    pltpu.sync_copy(ones_hbm, ones_v)
    pltpu.sync_copy(dst_hbm.at[pl.ds((c * NT + t) * K, K)], dvm)
    plsc.subcore_barrier()

    def chunk(k, carry):
        pltpu.sync_copy(ones_v, acc_sh.at[dvm.at[k]], add=True)
        return carry

    lax.fori_loop(0, K, chunk, 0)
    plsc.subcore_barrier()
    pltpu.sync_copy(acc_sh.at[pl.ds(r0, RPT)],
                    out_hbm.at[pl.ds(c * NP + r0, RPT)])


def _tc_prep(x_all, deg_hist, W1):
    def body(x_ref, dh_ref, w_ref, dinv_ref, hs_ref):
        deg = dh_ref[:, 0:1] + 1.0
        dinv = lax.rsqrt(jnp.maximum(deg, 1e-12))
        h = jnp.dot(x_ref[...], w_ref[...], preferred_element_type=jnp.float32)
        hs_ref[...] = h * dinv
        dinv_ref[...] = dinv

    return pl.pallas_call(
        body,
        grid=(NC, NB),
        in_specs=[
            pl.BlockSpec((None, BLK, D), lambda g, i: (g, i, 0)),
            pl.BlockSpec((None, BLK, 16), lambda g, i: (g, i, 0)),
            pl.BlockSpec((D, F1), lambda g, i: (0, 0)),
        ],
        out_specs=[
            pl.BlockSpec((None, BLK, 1), lambda g, i: (g, i, 0)),
            pl.BlockSpec((None, BLK, F1), lambda g, i: (g, i, 0)),
        ],
        out_shape=[
            jax.ShapeDtypeStruct((NC, N, 1), jnp.float32),
            jax.ShapeDtypeStruct((NC, N, F1), jnp.float32),
        ],
    )(x_all, deg_hist, W1)


def _tc_layer(acc, hs, dinv, b, W, Fl, Fn):
    def body(a_ref, h_ref, d_ref, b_ref, w_ref, o_ref):
        dv = d_ref[...]
        a = dv * (a_ref[...] + h_ref[...]) + b_ref[...]
        o = jnp.maximum(a, 0.0)
        o_ref[...] = jnp.dot(
            o, w_ref[...], preferred_element_type=jnp.float32) * dv

    return pl.pallas_call(
        body,
        grid=(NC, NB),
        in_specs=[
            pl.BlockSpec((None, BLK, Fl), lambda g, i: (g, i, 0)),
            pl.BlockSpec((None, BLK, Fl), lambda g, i: (g, i, 0)),
            pl.BlockSpec((None, BLK, 1), lambda g, i: (g, i, 0)),
            pl.BlockSpec((1, Fl), lambda g, i: (0, 0)),
            pl.BlockSpec((Fl, Fn), lambda g, i: (0, 0)),
        ],
        out_specs=pl.BlockSpec((None, BLK, Fn), lambda g, i: (g, i, 0)),
        out_shape=jax.ShapeDtypeStruct((NC, N, Fn), jnp.float32),
    )(acc, hs, dinv, b.reshape(1, Fl), W)


def _tc_final_a(acc, hs, dinv, b3):
    def body(a_ref, h_ref, d_ref, b_ref, o_ref):
        o_ref[...] = d_ref[...] * (a_ref[...] + h_ref[...]) + b_ref[...]

    return pl.pallas_call(
        body,
        grid=(NC, NB),
        in_specs=[
            pl.BlockSpec((None, BLK, F3), lambda g, i: (g, i, 0)),
            pl.BlockSpec((None, BLK, F3), lambda g, i: (g, i, 0)),
            pl.BlockSpec((None, BLK, 1), lambda g, i: (g, i, 0)),
            pl.BlockSpec((1, F3), lambda g, i: (0, 0)),
        ],
        out_specs=pl.BlockSpec((None, BLK, F3), lambda g, i: (g, i, 0)),
        out_shape=jax.ShapeDtypeStruct((NC, N, F3), jnp.float32),
    )(acc, hs, dinv, b3.reshape(1, F3))


def _tc_head(a_all, Wa, WtT, WblockT, bt, Wfc, bfc, Wsc, bsc):
    def body(a_ref, wa_ref, wt_ref, wb_ref, bt_ref, wfc_ref, bfc_ref,
             wsc_ref, bsc_ref, o_ref):
        ps = []
        for g in range(NC):
            ag = a_ref[g]
            mean = jnp.sum(ag, axis=0, keepdims=True) * (1.0 / N)
            tg = jnp.tanh(jnp.dot(mean, wa_ref[...],
                                  preferred_element_type=jnp.float32))
            coefs = jax.nn.sigmoid(jnp.sum(ag * tg, axis=1, keepdims=True))
            ps.append(jnp.sum(coefs * ag, axis=0, keepdims=True))
        p1, p2 = ps
        slices = []
        for t in range(T):
            v = jnp.dot(p1, wt_ref[t], preferred_element_type=jnp.float32)
            slices.append(jnp.sum(v * p2, axis=1, keepdims=True))
        scoring = jnp.concatenate(slices, axis=1)
        combined = jnp.concatenate([p1, p2], axis=1)
        block = jnp.dot(combined, wb_ref[...],
                        preferred_element_type=jnp.float32)
        s = jnp.maximum(scoring + block + bt_ref[...], 0.0)
        s = jnp.maximum(
            jnp.dot(s, wfc_ref[...], preferred_element_type=jnp.float32)
            + bfc_ref[...], 0.0)
        o_ref[...] = jax.nn.sigmoid(
            jnp.dot(s, wsc_ref[...], preferred_element_type=jnp.float32)
            + bsc_ref[...])

    return pl.pallas_call(
        body,
        out_shape=jax.ShapeDtypeStruct((1, 1), jnp.float32),
    )(a_all, Wa, WtT, WblockT, bt.reshape(1, T), Wfc, bfc.reshape(1, BN),
      Wsc, bsc.reshape(1, 1))


def kernel(x1, edge_index1, batch1, x2, edge_index2, batch2,
           W1, b1, W2, b2, W3, b3, Wa, Wt, Wblock, bt, Wfc, bfc, Wsc, bsc):
    del batch1, batch2
    pad = NT * K * CH - E
    i32 = jnp.int32
    zp = jnp.zeros((pad,), i32)
    s1 = jnp.concatenate([edge_index1[0], zp])
    s2 = jnp.concatenate([edge_index2[0], zp]) + N
    src_all = jnp.concatenate([s1, s2]).reshape(NC * NT * K, CH)
    dp = jnp.full((pad,), N, i32)
    d1 = jnp.concatenate([edge_index1[1], dp])
    d2 = jnp.concatenate([edge_index2[1], dp])
    dst_all = jnp.concatenate([d1, d2]).reshape(NC * NT * K, CH)
    x_all = jnp.stack([x1, x2])

    f32 = jnp.float32
    z16 = jnp.zeros((NP, 16), f32)
    z32 = jnp.zeros((NP, F2), f32)
    z64 = jnp.zeros((NP, F1), f32)
    ones16 = jnp.ones((CH, 16), f32)

    deg_hist = _sc_degree(dst_all, z16, ones16).reshape(NC, NP, 16)
    dinv, hs1 = _tc_prep(x_all, deg_hist, W1)
    acc1 = _sc_scatter(F1)(src_all, dst_all, hs1.reshape(NC * N, F1),
                           z64).reshape(NC, NP, F1)
    hs2 = _tc_layer(acc1, hs1, dinv, b1, W2, F1, F2)
    acc2 = _sc_scatter(F2)(src_all, dst_all, hs2.reshape(NC * N, F2),
                           z32).reshape(NC, NP, F2)
    hs3 = _tc_layer(acc2, hs2, dinv, b2, W3, F2, F3)
    acc3 = _sc_scatter(F3)(src_all, dst_all, hs3.reshape(NC * N, F3),
                           z16).reshape(NC, NP, F3)
    a_all = _tc_final_a(acc3, hs3, dinv, b3)
    score = _tc_head(a_all, Wa, jnp.transpose(Wt, (2, 0, 1)),
                     jnp.transpose(Wblock), bt, Wfc, bfc, Wsc, bsc)
    return score.reshape(-1)


# speedup vs baseline: 24.0499x; 1.3967x over previous
import functools

import jax
import jax.numpy as jnp
from jax import lax
from jax.experimental import pallas as pl
from jax.experimental.pallas import tpu as pltpu
from jax.experimental.pallas import tpu_sc as plsc

N = 10000
E = 320000
D = 128
F1, F2, F3 = 64, 32, 16
T = 16
BN = 16
NC = 2
NT = 16
CH = 128
K = 160
EPAD = NC * NT * K * CH
RPT = 632
NP = NT * RPT
NBUF = 4
BLK = 2000
NB = N // BLK

_mesh = plsc.VectorSubcoreMesh(core_axis_name="c", subcore_axis_name="s")


def _sc_scatter(F):
    @functools.partial(
        pl.kernel,
        out_type=jax.ShapeDtypeStruct((NC * NP, F), jnp.float32),
        mesh=_mesh,
        compiler_params=pltpu.CompilerParams(use_tc_tiling_on_sc=False),
        scratch_types=[
            pltpu.VMEM_SHARED((NP, F), jnp.float32),
            pltpu.VMEM((K, CH), jnp.int32),
            pltpu.VMEM((K, CH), jnp.int32),
            pltpu.VMEM((NBUF, CH, F), jnp.float32),
            pltpu.SemaphoreType.DMA,
        ],
    )
    def body(src_hbm, dst_hbm, hs_hbm, zeros_hbm, out_hbm, acc_sh, svm, dvm,
             rows, gsem):
        c = lax.axis_index("c")
        t = lax.axis_index("s")
        r0 = t * RPT
        pltpu.sync_copy(zeros_hbm.at[pl.ds(r0, RPT)], acc_sh.at[pl.ds(r0, RPT)])
        eb = (c * NT + t) * K
        pltpu.sync_copy(src_hbm.at[pl.ds(eb, K)], svm)
        pltpu.sync_copy(dst_hbm.at[pl.ds(eb, K)], dvm)
        plsc.subcore_barrier()

        for b in range(NBUF):
            pltpu.async_copy(hs_hbm.at[svm.at[b]], rows.at[b], gsem)

        def outer(kk, carry):
            for b in range(NBUF):
                k = kk * NBUF + b
                pltpu.make_async_copy(hs_hbm.at[svm.at[k]], rows.at[b],
                                      gsem).wait()
                pltpu.sync_copy(rows.at[b], acc_sh.at[dvm.at[k]], add=True)

                @pl.when(k + NBUF < K)
                def _():
                    pltpu.async_copy(hs_hbm.at[svm.at[k + NBUF]], rows.at[b],
                                     gsem)
            return carry

        lax.fori_loop(0, K // NBUF, outer, 0)
        plsc.subcore_barrier()
        pltpu.sync_copy(acc_sh.at[pl.ds(r0, RPT)],
                        out_hbm.at[pl.ds(c * NP + r0, RPT)])

    return body


@functools.partial(
    pl.kernel,
    out_type=jax.ShapeDtypeStruct((NC * NP, 16), jnp.float32),
    mesh=_mesh,
    compiler_params=pltpu.CompilerParams(use_tc_tiling_on_sc=False),
    scratch_types=[
        pltpu.VMEM_SHARED((NP, 16), jnp.float32),
        pltpu.VMEM((K, CH), jnp.int32),
        pltpu.VMEM((CH, 16), jnp.float32),
    ],
)
def _sc_degree(dst_hbm, zeros_hbm, ones_hbm, out_hbm, acc_sh, dvm, ones_v):
    c = lax.axis_index("c")
    t = lax.axis_index("s")
    r0 = t * RPT
    pltpu.sync_copy(zeros_hbm.at[pl.ds(r0, RPT)], acc_sh.at[pl.ds(r0, RPT)])
    pltpu.sync_copy(ones_hbm, ones_v)
    pltpu.sync_copy(dst_hbm.at[pl.ds((c * NT + t) * K, K)], dvm)
    plsc.subcore_barrier()

    def chunk(k, carry):
        pltpu.sync_copy(ones_v, acc_sh.at[dvm.at[k]], add=True)
        return carry

    lax.fori_loop(0, K, chunk, 0)
    plsc.subcore_barrier()
    pltpu.sync_copy(acc_sh.at[pl.ds(r0, RPT)],
                    out_hbm.at[pl.ds(c * NP + r0, RPT)])


def _tc_prep(x_all, deg_hist, W1):
    def body(x_ref, dh_ref, w_ref, dinv_ref, hs_ref):
        deg = dh_ref[:, 0:1] + 1.0
        dinv = lax.rsqrt(jnp.maximum(deg, 1e-12))
        h = jnp.dot(x_ref[...], w_ref[...], preferred_element_type=jnp.float32)
        hs_ref[...] = h * dinv
        dinv_ref[...] = dinv

    return pl.pallas_call(
        body,
        grid=(NC, NB),
        in_specs=[
            pl.BlockSpec((None, BLK, D), lambda g, i: (g, i, 0)),
            pl.BlockSpec((None, BLK, 16), lambda g, i: (g, i, 0)),
            pl.BlockSpec((D, F1), lambda g, i: (0, 0)),
        ],
        out_specs=[
            pl.BlockSpec((None, BLK, 1), lambda g, i: (g, i, 0)),
            pl.BlockSpec((None, BLK, F1), lambda g, i: (g, i, 0)),
        ],
        out_shape=[
            jax.ShapeDtypeStruct((NC, N, 1), jnp.float32),
            jax.ShapeDtypeStruct((NC, N, F1), jnp.float32),
        ],
    )(x_all, deg_hist, W1)


def _tc_layer(acc, hs, dinv, b, W, Fl, Fn):
    def body(a_ref, h_ref, d_ref, b_ref, w_ref, o_ref):
        dv = d_ref[...]
        a = dv * (a_ref[...] + h_ref[...]) + b_ref[...]
        o = jnp.maximum(a, 0.0)
        o_ref[...] = jnp.dot(
            o, w_ref[...], preferred_element_type=jnp.float32) * dv

    return pl.pallas_call(
        body,
        grid=(NC, NB),
        in_specs=[
            pl.BlockSpec((None, BLK, Fl), lambda g, i: (g, i, 0)),
            pl.BlockSpec((None, BLK, Fl), lambda g, i: (g, i, 0)),
            pl.BlockSpec((None, BLK, 1), lambda g, i: (g, i, 0)),
            pl.BlockSpec((1, Fl), lambda g, i: (0, 0)),
            pl.BlockSpec((Fl, Fn), lambda g, i: (0, 0)),
        ],
        out_specs=pl.BlockSpec((None, BLK, Fn), lambda g, i: (g, i, 0)),
        out_shape=jax.ShapeDtypeStruct((NC, N, Fn), jnp.float32),
    )(acc, hs, dinv, b.reshape(1, Fl), W)


def _tc_final_a(acc, hs, dinv, b3):
    def body(a_ref, h_ref, d_ref, b_ref, o_ref):
        o_ref[...] = d_ref[...] * (a_ref[...] + h_ref[...]) + b_ref[...]

    return pl.pallas_call(
        body,
        grid=(NC, NB),
        in_specs=[
            pl.BlockSpec((None, BLK, F3), lambda g, i: (g, i, 0)),
            pl.BlockSpec((None, BLK, F3), lambda g, i: (g, i, 0)),
            pl.BlockSpec((None, BLK, 1), lambda g, i: (g, i, 0)),
            pl.BlockSpec((1, F3), lambda g, i: (0, 0)),
        ],
        out_specs=pl.BlockSpec((None, BLK, F3), lambda g, i: (g, i, 0)),
        out_shape=jax.ShapeDtypeStruct((NC, N, F3), jnp.float32),
    )(acc, hs, dinv, b3.reshape(1, F3))


def _tc_head(a_all, Wa, WtT, WblockT, bt, Wfc, bfc, Wsc, bsc):
    def body(a_ref, wa_ref, wt_ref, wb_ref, bt_ref, wfc_ref, bfc_ref,
             wsc_ref, bsc_ref, o_ref):
        ps = []
        for g in range(NC):
            ag = a_ref[g]
            mean = jnp.sum(ag, axis=0, keepdims=True) * (1.0 / N)
            tg = jnp.tanh(jnp.dot(mean, wa_ref[...],
                                  preferred_element_type=jnp.float32))
            coefs = jax.nn.sigmoid(jnp.sum(ag * tg, axis=1, keepdims=True))
            ps.append(jnp.sum(coefs * ag, axis=0, keepdims=True))
        p1, p2 = ps
        slices = []
        for t in range(T):
            v = jnp.dot(p1, wt_ref[t], preferred_element_type=jnp.float32)
            slices.append(jnp.sum(v * p2, axis=1, keepdims=True))
        scoring = jnp.concatenate(slices, axis=1)
        combined = jnp.concatenate([p1, p2], axis=1)
        block = jnp.dot(combined, wb_ref[...],
                        preferred_element_type=jnp.float32)
        s = jnp.maximum(scoring + block + bt_ref[...], 0.0)
        s = jnp.maximum(
            jnp.dot(s, wfc_ref[...], preferred_element_type=jnp.float32)
            + bfc_ref[...], 0.0)
        o_ref[...] = jax.nn.sigmoid(
            jnp.dot(s, wsc_ref[...], preferred_element_type=jnp.float32)
            + bsc_ref[...])

    return pl.pallas_call(
        body,
        out_shape=jax.ShapeDtypeStruct((1, 1), jnp.float32),
    )(a_all, Wa, WtT, WblockT, bt.reshape(1, T), Wfc, bfc.reshape(1, BN),
      Wsc, bsc.reshape(1, 1))


def kernel(x1, edge_index1, batch1, x2, edge_index2, batch2,
           W1, b1, W2, b2, W3, b3, Wa, Wt, Wblock, bt, Wfc, bfc, Wsc, bsc):
    del batch1, batch2
    pad = NT * K * CH - E
    i32 = jnp.int32
    zp = jnp.zeros((pad,), i32)
    s1 = jnp.concatenate([edge_index1[0], zp])
    s2 = jnp.concatenate([edge_index2[0], zp]) + N
    src_all = jnp.concatenate([s1, s2]).reshape(NC * NT * K, CH)
    dp = jnp.full((pad,), N, i32)
    d1 = jnp.concatenate([edge_index1[1], dp])
    d2 = jnp.concatenate([edge_index2[1], dp])
    dst_all = jnp.concatenate([d1, d2]).reshape(NC * NT * K, CH)
    x_all = jnp.stack([x1, x2])

    f32 = jnp.float32
    z16 = jnp.zeros((NP, 16), f32)
    z32 = jnp.zeros((NP, F2), f32)
    z64 = jnp.zeros((NP, F1), f32)
    ones16 = jnp.ones((CH, 16), f32)

    deg_hist = _sc_degree(dst_all, z16, ones16).reshape(NC, NP, 16)
    dinv, hs1 = _tc_prep(x_all, deg_hist, W1)
    acc1 = _sc_scatter(F1)(src_all, dst_all, hs1.reshape(NC * N, F1),
                           z64).reshape(NC, NP, F1)
    hs2 = _tc_layer(acc1, hs1, dinv, b1, W2, F1, F2)
    acc2 = _sc_scatter(F2)(src_all, dst_all, hs2.reshape(NC * N, F2),
                           z32).reshape(NC, NP, F2)
    hs3 = _tc_layer(acc2, hs2, dinv, b2, W3, F2, F3)
    acc3 = _sc_scatter(F3)(src_all, dst_all, hs3.reshape(NC * N, F3),
                           z16).reshape(NC, NP, F3)
    a_all = _tc_final_a(acc3, hs3, dinv, b3)
    score = _tc_head(a_all, Wa, jnp.transpose(Wt, (2, 0, 1)),
                     jnp.transpose(Wblock), bt, Wfc, bfc, Wsc, bsc)
    return score.reshape(-1)


# speedup vs baseline: 24.4528x; 1.0168x over previous
import functools

import jax
import jax.numpy as jnp
from jax import lax
from jax.experimental import pallas as pl
from jax.experimental.pallas import tpu as pltpu
from jax.experimental.pallas import tpu_sc as plsc

N = 10000
E = 320000
D = 128
F1, F2, F3 = 64, 32, 16
T = 16
BN = 16
NC = 2
NT = 16
CH = 128
K = 160
EPAD = NC * NT * K * CH
RPT = 632
NP = NT * RPT
_RING = {F1: (5, 3), F2: (8, 4), F3: (8, 4)}
BLK = 2000
NB = N // BLK

_mesh = plsc.VectorSubcoreMesh(core_axis_name="c", subcore_axis_name="s")


@functools.lru_cache(maxsize=None)
def _sc_scatter(F):
    NBUF, GAH = _RING[F]

    @functools.partial(
        pl.kernel,
        out_type=jax.ShapeDtypeStruct((NC * NP, F), jnp.float32),
        mesh=_mesh,
        compiler_params=pltpu.CompilerParams(use_tc_tiling_on_sc=False),
        scratch_types=[
            pltpu.VMEM_SHARED((NP, F), jnp.float32),
            pltpu.VMEM((K, CH), jnp.int32),
            pltpu.VMEM((K, CH), jnp.int32),
            pltpu.VMEM((NBUF, CH, F), jnp.float32),
            pltpu.SemaphoreType.DMA,
            pltpu.SemaphoreType.DMA,
        ],
    )
    def body(src_hbm, dst_hbm, hs_hbm, zeros_hbm, out_hbm, acc_sh, svm, dvm,
             rows, gsem, ssem):
        c = lax.axis_index("c")
        t = lax.axis_index("s")
        r0 = t * RPT
        pltpu.sync_copy(zeros_hbm.at[pl.ds(r0, RPT)], acc_sh.at[pl.ds(r0, RPT)])
        eb = (c * NT + t) * K
        pltpu.sync_copy(src_hbm.at[pl.ds(eb, K)], svm)
        pltpu.sync_copy(dst_hbm.at[pl.ds(eb, K)], dvm)
        plsc.subcore_barrier()

        for b in range(GAH):
            pltpu.async_copy(hs_hbm.at[svm.at[b]], rows.at[b], gsem)

        def outer(kk, carry):
            for j in range(NBUF):
                k = kk * NBUF + j
                pltpu.make_async_copy(hs_hbm.at[svm.at[k]], rows.at[j],
                                      gsem).wait()
                pltpu.async_copy(rows.at[j], acc_sh.at[dvm.at[k]], ssem,
                                 add=True)

                @pl.when(k >= NBUF - GAH)
                def _():
                    pltpu.make_async_copy(rows.at[j], acc_sh.at[dvm.at[k]],
                                          ssem).wait()

                @pl.when(k + GAH < K)
                def _():
                    pltpu.async_copy(hs_hbm.at[svm.at[k + GAH]],
                                     rows.at[(j + GAH) % NBUF], gsem)
            return carry

        lax.fori_loop(0, K // NBUF, outer, 0)
        for j in range(NBUF - GAH):
            pltpu.make_async_copy(rows.at[j], acc_sh.at[dvm.at[j]],
                                  ssem).wait()
        plsc.subcore_barrier()
        pltpu.sync_copy(acc_sh.at[pl.ds(r0, RPT)],
                        out_hbm.at[pl.ds(c * NP + r0, RPT)])

    return body


@functools.partial(
    pl.kernel,
    out_type=jax.ShapeDtypeStruct((NC * NP, 16), jnp.float32),
    mesh=_mesh,
    compiler_params=pltpu.CompilerParams(use_tc_tiling_on_sc=False),
    scratch_types=[
        pltpu.VMEM_SHARED((NP, 16), jnp.float32),
        pltpu.VMEM((K, CH), jnp.int32),
        pltpu.VMEM((CH, 16), jnp.float32),
    ],
)
def _sc_degree(dst_hbm, zeros_hbm, ones_hbm, out_hbm, acc_sh, dvm, ones_v):
    c = lax.axis_index("c")
    t = lax.axis_index("s")
    r0 = t * RPT
    pltpu.sync_copy(zeros_hbm.at[pl.ds(r0, RPT)], acc_sh.at[pl.ds(r0, RPT)])
    pltpu.sync_copy(ones_hbm, ones_v)
    pltpu.sync_copy(dst_hbm.at[pl.ds((c * NT + t) * K, K)], dvm)
    plsc.subcore_barrier()

    def chunk(k, carry):
        pltpu.sync_copy(ones_v, acc_sh.at[dvm.at[k]], add=True)
        return carry

    lax.fori_loop(0, K, chunk, 0)
    plsc.subcore_barrier()
    pltpu.sync_copy(acc_sh.at[pl.ds(r0, RPT)],
                    out_hbm.at[pl.ds(c * NP + r0, RPT)])


def _tc_prep(x_all, deg_hist, W1):
    def body(x_ref, dh_ref, w_ref, dinv_ref, hs_ref):
        deg = dh_ref[:, 0:1] + 1.0
        dinv = lax.rsqrt(jnp.maximum(deg, 1e-12))
        h = jnp.dot(x_ref[...], w_ref[...], preferred_element_type=jnp.float32)
        hs_ref[...] = h * dinv
        dinv_ref[...] = dinv

    return pl.pallas_call(
        body,
        grid=(NC, NB),
        in_specs=[
            pl.BlockSpec((None, BLK, D), lambda g, i: (g, i, 0)),
            pl.BlockSpec((None, BLK, 16), lambda g, i: (g, i, 0)),
            pl.BlockSpec((D, F1), lambda g, i: (0, 0)),
        ],
        out_specs=[
            pl.BlockSpec((None, BLK, 1), lambda g, i: (g, i, 0)),
            pl.BlockSpec((None, BLK, F1), lambda g, i: (g, i, 0)),
        ],
        out_shape=[
            jax.ShapeDtypeStruct((NC, N, 1), jnp.float32),
            jax.ShapeDtypeStruct((NC, N, F1), jnp.float32),
        ],
    )(x_all, deg_hist, W1)


def _tc_layer(acc, hs, dinv, b, W, Fl, Fn):
    def body(a_ref, h_ref, d_ref, b_ref, w_ref, o_ref):
        dv = d_ref[...]
        a = dv * (a_ref[...] + h_ref[...]) + b_ref[...]
        o = jnp.maximum(a, 0.0)
        o_ref[...] = jnp.dot(
            o, w_ref[...], preferred_element_type=jnp.float32) * dv

    return pl.pallas_call(
        body,
        grid=(NC, NB),
        in_specs=[
            pl.BlockSpec((None, BLK, Fl), lambda g, i: (g, i, 0)),
            pl.BlockSpec((None, BLK, Fl), lambda g, i: (g, i, 0)),
            pl.BlockSpec((None, BLK, 1), lambda g, i: (g, i, 0)),
            pl.BlockSpec((1, Fl), lambda g, i: (0, 0)),
            pl.BlockSpec((Fl, Fn), lambda g, i: (0, 0)),
        ],
        out_specs=pl.BlockSpec((None, BLK, Fn), lambda g, i: (g, i, 0)),
        out_shape=jax.ShapeDtypeStruct((NC, N, Fn), jnp.float32),
    )(acc, hs, dinv, b.reshape(1, Fl), W)


def _tc_head(acc, hs, dinv, b3, Wa, WtT, WblockT, bt, Wfc, bfc, Wsc, bsc):
    def body(acc_ref, hs_ref, d_ref, b3_ref, wa_ref, wt_ref, wb_ref, bt_ref,
             wfc_ref, bfc_ref, wsc_ref, bsc_ref, o_ref):
        ps = []
        for g in range(NC):
            ag = (d_ref[g] * (acc_ref[g, 0:N, :] + hs_ref[g])
                  + b3_ref[...])
            mean = jnp.sum(ag, axis=0, keepdims=True) * (1.0 / N)
            tg = jnp.tanh(jnp.dot(mean, wa_ref[...],
                                  preferred_element_type=jnp.float32))
            coefs = jax.nn.sigmoid(jnp.sum(ag * tg, axis=1, keepdims=True))
            ps.append(jnp.sum(coefs * ag, axis=0, keepdims=True))
        p1, p2 = ps
        slices = []
        for t in range(T):
            v = jnp.dot(p1, wt_ref[t], preferred_element_type=jnp.float32)
            slices.append(jnp.sum(v * p2, axis=1, keepdims=True))
        scoring = jnp.concatenate(slices, axis=1)
        combined = jnp.concatenate([p1, p2], axis=1)
        block = jnp.dot(combined, wb_ref[...],
                        preferred_element_type=jnp.float32)
        s = jnp.maximum(scoring + block + bt_ref[...], 0.0)
        s = jnp.maximum(
            jnp.dot(s, wfc_ref[...], preferred_element_type=jnp.float32)
            + bfc_ref[...], 0.0)
        o_ref[...] = jax.nn.sigmoid(
            jnp.dot(s, wsc_ref[...], preferred_element_type=jnp.float32)
            + bsc_ref[...])

    return pl.pallas_call(
        body,
        out_shape=jax.ShapeDtypeStruct((1, 1), jnp.float32),
    )(acc, hs, dinv, b3.reshape(1, F3), Wa, WtT, WblockT, bt.reshape(1, T),
      Wfc, bfc.reshape(1, BN), Wsc, bsc.reshape(1, 1))


def kernel(x1, edge_index1, batch1, x2, edge_index2, batch2,
           W1, b1, W2, b2, W3, b3, Wa, Wt, Wblock, bt, Wfc, bfc, Wsc, bsc):
    del batch1, batch2
    pad = NT * K * CH - E
    i32 = jnp.int32
    zp = jnp.zeros((pad,), i32)
    s1 = jnp.concatenate([edge_index1[0], zp])
    s2 = jnp.concatenate([edge_index2[0], zp]) + N
    src_all = jnp.concatenate([s1, s2]).reshape(NC * NT * K, CH)
    dp = jnp.full((pad,), N, i32)
    d1 = jnp.concatenate([edge_index1[1], dp])
    d2 = jnp.concatenate([edge_index2[1], dp])
    dst_all = jnp.concatenate([d1, d2]).reshape(NC * NT * K, CH)
    x_all = jnp.stack([x1, x2])

    f32 = jnp.float32
    z16 = jnp.zeros((NP, 16), f32)
    z32 = jnp.zeros((NP, F2), f32)
    z64 = jnp.zeros((NP, F1), f32)
    ones16 = jnp.ones((CH, 16), f32)

    deg_hist = _sc_degree(dst_all, z16, ones16).reshape(NC, NP, 16)
    dinv, hs1 = _tc_prep(x_all, deg_hist, W1)
    acc1 = _sc_scatter(F1)(src_all, dst_all, hs1.reshape(NC * N, F1),
                           z64).reshape(NC, NP, F1)
    hs2 = _tc_layer(acc1, hs1, dinv, b1, W2, F1, F2)
    acc2 = _sc_scatter(F2)(src_all, dst_all, hs2.reshape(NC * N, F2),
                           z32).reshape(NC, NP, F2)
    hs3 = _tc_layer(acc2, hs2, dinv, b2, W3, F2, F3)
    acc3 = _sc_scatter(F3)(src_all, dst_all, hs3.reshape(NC * N, F3),
                           z16).reshape(NC, NP, F3)
    score = _tc_head(acc3, hs3, dinv, b3, Wa, jnp.transpose(Wt, (2, 0, 1)),
                     jnp.transpose(Wblock), bt, Wfc, bfc, Wsc, bsc)
    return score.reshape(-1)


# speedup vs baseline: 24.5155x; 1.0026x over previous
import functools

import jax
import jax.numpy as jnp
from jax import lax
from jax.experimental import pallas as pl
from jax.experimental.pallas import tpu as pltpu
from jax.experimental.pallas import tpu_sc as plsc

N = 10000
E = 320000
D = 128
F1, F2, F3 = 64, 32, 16
T = 16
BN = 16
NC = 2
NT = 16
CH = 128
K = 160
EPAD = NC * NT * K * CH
RPT = 632
NP = NT * RPT
_RING = {F1: (5, 3), F2: (8, 4), F3: (8, 4)}
BLK = 2000
NB = N // BLK

_mesh = plsc.VectorSubcoreMesh(core_axis_name="c", subcore_axis_name="s")


@functools.lru_cache(maxsize=None)
def _sc_scatter(F):
    NBUF, GAH = _RING[F]

    @functools.partial(
        pl.kernel,
        out_type=jax.ShapeDtypeStruct((NC * NP, F), jnp.float32),
        mesh=_mesh,
        compiler_params=pltpu.CompilerParams(use_tc_tiling_on_sc=False),
        scratch_types=[
            pltpu.VMEM_SHARED((NP, F), jnp.float32),
            pltpu.VMEM((K, CH), jnp.int32),
            pltpu.VMEM((K, CH), jnp.int32),
            pltpu.VMEM((NBUF, CH, F), jnp.float32),
            pltpu.SemaphoreType.DMA,
            pltpu.SemaphoreType.DMA,
        ],
    )
    def body(src_hbm, dst_hbm, hs_hbm, zeros_hbm, out_hbm, acc_sh, svm, dvm,
             rows, gsem, ssem):
        c = lax.axis_index("c")
        t = lax.axis_index("s")
        r0 = t * RPT
        pltpu.sync_copy(zeros_hbm.at[pl.ds(r0, RPT)], acc_sh.at[pl.ds(r0, RPT)])
        eb = (c * NT + t) * K
        pltpu.sync_copy(src_hbm.at[pl.ds(eb, K)], svm)
        pltpu.sync_copy(dst_hbm.at[pl.ds(eb, K)], dvm)
        plsc.subcore_barrier()

        for b in range(GAH):
            pltpu.async_copy(hs_hbm.at[svm.at[b]], rows.at[b], gsem)

        def outer(kk, carry):
            for j in range(NBUF):
                k = kk * NBUF + j
                pltpu.make_async_copy(hs_hbm.at[svm.at[k]], rows.at[j],
                                      gsem).wait()
                pltpu.async_copy(rows.at[j], acc_sh.at[dvm.at[k]], ssem,
                                 add=True)

                @pl.when(k >= NBUF - GAH)
                def _():
                    pltpu.make_async_copy(rows.at[j], acc_sh.at[dvm.at[k]],
                                          ssem).wait()

                @pl.when(k + GAH < K)
                def _():
                    pltpu.async_copy(hs_hbm.at[svm.at[k + GAH]],
                                     rows.at[(j + GAH) % NBUF], gsem)
            return carry

        lax.fori_loop(0, K // NBUF, outer, 0)
        for j in range(NBUF - GAH):
            pltpu.make_async_copy(rows.at[j], acc_sh.at[dvm.at[j]],
                                  ssem).wait()
        plsc.subcore_barrier()
        pltpu.sync_copy(acc_sh.at[pl.ds(r0, RPT)],
                        out_hbm.at[pl.ds(c * NP + r0, RPT)])

    return body


@functools.partial(
    pl.kernel,
    out_type=jax.ShapeDtypeStruct((NC * NP, 16), jnp.float32),
    mesh=_mesh,
    compiler_params=pltpu.CompilerParams(use_tc_tiling_on_sc=False),
    scratch_types=[
        pltpu.VMEM_SHARED((NP, 16), jnp.float32),
        pltpu.VMEM((K, CH), jnp.int32),
        pltpu.VMEM((CH, 16), jnp.float32),
    ],
)
def _sc_degree(dst_hbm, zeros_hbm, ones_hbm, out_hbm, acc_sh, dvm, ones_v):
    c = lax.axis_index("c")
    t = lax.axis_index("s")
    r0 = t * RPT
    pltpu.sync_copy(zeros_hbm.at[pl.ds(r0, RPT)], acc_sh.at[pl.ds(r0, RPT)])
    pltpu.sync_copy(ones_hbm, ones_v)
    pltpu.sync_copy(dst_hbm.at[pl.ds((c * NT + t) * K, K)], dvm)
    plsc.subcore_barrier()

    def chunk(k, carry):
        pltpu.sync_copy(ones_v, acc_sh.at[dvm.at[k]], add=True)
        return carry

    lax.fori_loop(0, K, chunk, 0)
    plsc.subcore_barrier()
    pltpu.sync_copy(acc_sh.at[pl.ds(r0, RPT)],
                    out_hbm.at[pl.ds(c * NP + r0, RPT)])


def _tc_matmul1(x_all, W1):
    def body(x_ref, w_ref, h_ref):
        h_ref[...] = jnp.dot(x_ref[...], w_ref[...],
                             preferred_element_type=jnp.float32)

    return pl.pallas_call(
        body,
        grid=(NC, NB),
        in_specs=[
            pl.BlockSpec((None, BLK, D), lambda g, i: (g, i, 0)),
            pl.BlockSpec((D, F1), lambda g, i: (0, 0)),
        ],
        out_specs=pl.BlockSpec((None, BLK, F1), lambda g, i: (g, i, 0)),
        out_shape=jax.ShapeDtypeStruct((NC, N, F1), jnp.float32),
    )(x_all, W1)


def _tc_prep(h1, deg_hist):
    def body(h_ref, dh_ref, dinv_ref, hs_ref):
        deg = dh_ref[:, 0:1] + 1.0
        dinv = lax.rsqrt(jnp.maximum(deg, 1e-12))
        hs_ref[...] = h_ref[...] * dinv
        dinv_ref[...] = dinv

    return pl.pallas_call(
        body,
        grid=(NC, NB),
        in_specs=[
            pl.BlockSpec((None, BLK, F1), lambda g, i: (g, i, 0)),
            pl.BlockSpec((None, BLK, 16), lambda g, i: (g, i, 0)),
        ],
        out_specs=[
            pl.BlockSpec((None, BLK, 1), lambda g, i: (g, i, 0)),
            pl.BlockSpec((None, BLK, F1), lambda g, i: (g, i, 0)),
        ],
        out_shape=[
            jax.ShapeDtypeStruct((NC, N, 1), jnp.float32),
            jax.ShapeDtypeStruct((NC, N, F1), jnp.float32),
        ],
    )(h1, deg_hist)


def _tc_layer(acc, hs, dinv, b, W, Fl, Fn):
    def body(a_ref, h_ref, d_ref, b_ref, w_ref, o_ref):
        dv = d_ref[...]
        a = dv * (a_ref[...] + h_ref[...]) + b_ref[...]
        o = jnp.maximum(a, 0.0)
        o_ref[...] = jnp.dot(
            o, w_ref[...], preferred_element_type=jnp.float32) * dv

    return pl.pallas_call(
        body,
        grid=(NC, NB),
        in_specs=[
            pl.BlockSpec((None, BLK, Fl), lambda g, i: (g, i, 0)),
            pl.BlockSpec((None, BLK, Fl), lambda g, i: (g, i, 0)),
            pl.BlockSpec((None, BLK, 1), lambda g, i: (g, i, 0)),
            pl.BlockSpec((1, Fl), lambda g, i: (0, 0)),
            pl.BlockSpec((Fl, Fn), lambda g, i: (0, 0)),
        ],
        out_specs=pl.BlockSpec((None, BLK, Fn), lambda g, i: (g, i, 0)),
        out_shape=jax.ShapeDtypeStruct((NC, N, Fn), jnp.float32),
    )(acc, hs, dinv, b.reshape(1, Fl), W)


def _tc_head(acc, hs, dinv, b3, Wa, WtT, WblockT, bt, Wfc, bfc, Wsc, bsc):
    def body(acc_ref, hs_ref, d_ref, b3_ref, wa_ref, wt_ref, wb_ref, bt_ref,
             wfc_ref, bfc_ref, wsc_ref, bsc_ref, o_ref):
        ps = []
        for g in range(NC):
            ag = (d_ref[g] * (acc_ref[g, 0:N, :] + hs_ref[g])
                  + b3_ref[...])
            mean = jnp.sum(ag, axis=0, keepdims=True) * (1.0 / N)
            tg = jnp.tanh(jnp.dot(mean, wa_ref[...],
                                  preferred_element_type=jnp.float32))
            coefs = jax.nn.sigmoid(jnp.sum(ag * tg, axis=1, keepdims=True))
            ps.append(jnp.sum(coefs * ag, axis=0, keepdims=True))
        p1, p2 = ps
        slices = []
        for t in range(T):
            v = jnp.dot(p1, wt_ref[t], preferred_element_type=jnp.float32)
            slices.append(jnp.sum(v * p2, axis=1, keepdims=True))
        scoring = jnp.concatenate(slices, axis=1)
        combined = jnp.concatenate([p1, p2], axis=1)
        block = jnp.dot(combined, wb_ref[...],
                        preferred_element_type=jnp.float32)
        s = jnp.maximum(scoring + block + bt_ref[...], 0.0)
        s = jnp.maximum(
            jnp.dot(s, wfc_ref[...], preferred_element_type=jnp.float32)
            + bfc_ref[...], 0.0)
        o_ref[...] = jax.nn.sigmoid(
            jnp.dot(s, wsc_ref[...], preferred_element_type=jnp.float32)
            + bsc_ref[...])

    return pl.pallas_call(
        body,
        out_shape=jax.ShapeDtypeStruct((1, 1), jnp.float32),
    )(acc, hs, dinv, b3.reshape(1, F3), Wa, WtT, WblockT, bt.reshape(1, T),
      Wfc, bfc.reshape(1, BN), Wsc, bsc.reshape(1, 1))


def kernel(x1, edge_index1, batch1, x2, edge_index2, batch2,
           W1, b1, W2, b2, W3, b3, Wa, Wt, Wblock, bt, Wfc, bfc, Wsc, bsc):
    del batch1, batch2
    pad = NT * K * CH - E
    i32 = jnp.int32
    zp = jnp.zeros((pad,), i32)
    s1 = jnp.concatenate([edge_index1[0], zp])
    s2 = jnp.concatenate([edge_index2[0], zp]) + N
    src_all = jnp.concatenate([s1, s2]).reshape(NC * NT * K, CH)
    dp = jnp.full((pad,), N, i32)
    d1 = jnp.concatenate([edge_index1[1], dp])
    d2 = jnp.concatenate([edge_index2[1], dp])
    dst_all = jnp.concatenate([d1, d2]).reshape(NC * NT * K, CH)
    x_all = jnp.stack([x1, x2])

    f32 = jnp.float32
    z16 = jnp.zeros((NP, 16), f32)
    z32 = jnp.zeros((NP, F2), f32)
    z64 = jnp.zeros((NP, F1), f32)
    ones16 = jnp.ones((CH, 16), f32)

    deg_hist = _sc_degree(dst_all, z16, ones16).reshape(NC, NP, 16)
    h1 = _tc_matmul1(x_all, W1)
    dinv, hs1 = _tc_prep(h1, deg_hist)
    acc1 = _sc_scatter(F1)(src_all, dst_all, hs1.reshape(NC * N, F1),
                           z64).reshape(NC, NP, F1)
    hs2 = _tc_layer(acc1, hs1, dinv, b1, W2, F1, F2)
    acc2 = _sc_scatter(F2)(src_all, dst_all, hs2.reshape(NC * N, F2),
                           z32).reshape(NC, NP, F2)
    hs3 = _tc_layer(acc2, hs2, dinv, b2, W3, F2, F3)
    acc3 = _sc_scatter(F3)(src_all, dst_all, hs3.reshape(NC * N, F3),
                           z16).reshape(NC, NP, F3)
    score = _tc_head(acc3, hs3, dinv, b3, Wa, jnp.transpose(Wt, (2, 0, 1)),
                     jnp.transpose(Wblock), bt, Wfc, bfc, Wsc, bsc)
    return score.reshape(-1)
